# Initial kernel scaffold; baseline (speedup 1.0000x reference)
import jax
import jax.numpy as jnp
from jax.experimental import pallas as pl


def kernel(x, edge_index, W1, b1, W2, b2):
    raise NotImplementedError("write your pallas kernel here")


# speedup vs baseline: 14.3924x; 14.3924x over previous
import functools

import jax
import jax.numpy as jnp
from jax import lax
from jax.experimental import pallas as pl
from jax.experimental.pallas import tpu as pltpu
from jax.experimental.pallas import tpu_sc as plsc

N = 10000
E = 320000
DIN = 128
DHID = 128
DOUT = 16

NC = 2
NS = 16
NW = NC * NS
C = 128
EPW = 10112
EP = NW * EPW
NCHUNK = EPW // C
NA = 10240
RZ = NA // NS
RD = NA // NS

_MESH = plsc.VectorSubcoreMesh(core_axis_name="c", subcore_axis_name="s")


def _make_edge_scatter(D):
    @functools.partial(
        pl.kernel,
        mesh=_MESH,
        compiler_params=pltpu.CompilerParams(use_tc_tiling_on_sc=(D % 128 == 0)),
        out_type=jax.ShapeDtypeStruct((NC, NA, D), jnp.float32),
        scratch_types=[
            pltpu.VMEM((C,), jnp.int32),
            pltpu.VMEM((C,), jnp.int32),
            pltpu.VMEM((C, D), jnp.float32),
            pltpu.VMEM_SHARED((NA, D), jnp.float32),
        ],
    )
    def k(y_hbm, src_hbm, dst_hbm, z_hbm, out_hbm, srcv, dstv, rows, acc):
        cid = lax.axis_index("c")
        sid = lax.axis_index("s")
        w = sid * NC + cid
        pltpu.sync_copy(z_hbm, acc.at[pl.ds(sid * RZ, RZ)])
        plsc.subcore_barrier()

        def body(g, carry):
            base = w * EPW + g * C
            pltpu.sync_copy(src_hbm.at[pl.ds(base, C)], srcv)
            pltpu.sync_copy(dst_hbm.at[pl.ds(base, C)], dstv)
            pltpu.sync_copy(y_hbm.at[srcv], rows)
            pltpu.sync_copy(rows, acc.at[dstv], add=True)
            return carry

        lax.fori_loop(0, NCHUNK, body, 0)
        plsc.subcore_barrier()
        pltpu.sync_copy(
            acc.at[pl.ds(sid * RD, RD)], out_hbm.at[cid, pl.ds(sid * RD, RD)]
        )

    return k


@functools.partial(
    pl.kernel,
    mesh=_MESH,
    compiler_params=pltpu.CompilerParams(use_tc_tiling_on_sc=False),
    out_type=jax.ShapeDtypeStruct((NC, NA, DOUT), jnp.float32),
    scratch_types=[
        pltpu.VMEM((C,), jnp.int32),
        pltpu.VMEM((C, DOUT), jnp.float32),
        pltpu.VMEM_SHARED((NA, DOUT), jnp.float32),
    ],
)
def _deg_kernel(dst_hbm, ones_hbm, z_hbm, out_hbm, dstv, onesv, acc):
    cid = lax.axis_index("c")
    sid = lax.axis_index("s")
    w = sid * NC + cid
    pltpu.sync_copy(z_hbm, acc.at[pl.ds(sid * RZ, RZ)])
    pltpu.sync_copy(ones_hbm, onesv)
    plsc.subcore_barrier()

    def body(g, carry):
        base = w * EPW + g * C
        pltpu.sync_copy(dst_hbm.at[pl.ds(base, C)], dstv)
        pltpu.sync_copy(onesv, acc.at[dstv], add=True)
        return carry

    lax.fori_loop(0, NCHUNK, body, 0)
    plsc.subcore_barrier()
    pltpu.sync_copy(acc.at[pl.ds(sid * RD, RD)], out_hbm.at[cid, pl.ds(sid * RD, RD)])


_edge_scatter_128 = _make_edge_scatter(DHID)
_edge_scatter_16 = _make_edge_scatter(DOUT)

BR = 1000


def _dinv_block(degp_ref):
    deg = degp_ref[0, :, 0:1] + degp_ref[1, :, 0:1] + 1.0
    return lax.rsqrt(deg)


def _k1_body(degp_ref, x_ref, w1_ref, y_ref):
    dinv = _dinv_block(degp_ref)
    xw = jnp.dot(x_ref[...], w1_ref[...], preferred_element_type=jnp.float32)
    y_ref[...] = xw * dinv


def _k3_body(degp_ref, s1p_ref, y1_ref, b1_ref, w2_ref, y2_ref):
    dinv = _dinv_block(degp_ref)
    h = jnp.maximum(dinv * (s1p_ref[0] + s1p_ref[1] + y1_ref[...]) + b1_ref[...], 0.0)
    y2_ref[...] = (
        jnp.dot(h, w2_ref[...], preferred_element_type=jnp.float32) * dinv
    )


def _k5_body(degp_ref, s2p_ref, y2_ref, b2_ref, o_ref):
    dinv = _dinv_block(degp_ref)
    o_ref[...] = dinv * (s2p_ref[0] + s2p_ref[1] + y2_ref[...]) + b2_ref[...]


def _degp_spec():
    return pl.BlockSpec((2, BR, DOUT), lambda i: (0, i, 0))


def kernel(x, edge_index, W1, b1, W2, b2):
    src = edge_index[0]
    dst = edge_index[1]
    pad = EP - E
    srcp = jnp.concatenate([src, jnp.zeros((pad,), jnp.int32)])
    dstp = jnp.concatenate([dst, jnp.full((pad,), N, jnp.int32)])
    ones16 = jnp.ones((C, DOUT), jnp.float32)
    z16 = jnp.zeros((RZ, DOUT), jnp.float32)
    z128 = jnp.zeros((RZ, DHID), jnp.float32)

    deg_p = _deg_kernel(dstp, ones16, z16)

    y1 = pl.pallas_call(
        _k1_body,
        grid=(N // BR,),
        in_specs=[
            _degp_spec(),
            pl.BlockSpec((BR, DIN), lambda i: (i, 0)),
            pl.BlockSpec((DIN, DHID), lambda i: (0, 0)),
        ],
        out_specs=pl.BlockSpec((BR, DHID), lambda i: (i, 0)),
        out_shape=jax.ShapeDtypeStruct((N, DHID), jnp.float32),
    )(deg_p, x, W1)

    s1p = _edge_scatter_128(y1, srcp, dstp, z128)

    y2 = pl.pallas_call(
        _k3_body,
        grid=(N // BR,),
        in_specs=[
            _degp_spec(),
            pl.BlockSpec((2, BR, DHID), lambda i: (0, i, 0)),
            pl.BlockSpec((BR, DHID), lambda i: (i, 0)),
            pl.BlockSpec((1, DHID), lambda i: (0, 0)),
            pl.BlockSpec((DHID, DOUT), lambda i: (0, 0)),
        ],
        out_specs=pl.BlockSpec((BR, DOUT), lambda i: (i, 0)),
        out_shape=jax.ShapeDtypeStruct((N, DOUT), jnp.float32),
    )(deg_p, s1p, y1, b1.reshape(1, DHID), W2)

    s2p = _edge_scatter_16(y2, srcp, dstp, z16)

    out = pl.pallas_call(
        _k5_body,
        grid=(N // BR,),
        in_specs=[
            _degp_spec(),
            pl.BlockSpec((2, BR, DOUT), lambda i: (0, i, 0)),
            pl.BlockSpec((BR, DOUT), lambda i: (i, 0)),
            pl.BlockSpec((1, DOUT), lambda i: (0, 0)),
        ],
        out_specs=pl.BlockSpec((BR, DOUT), lambda i: (i, 0)),
        out_shape=jax.ShapeDtypeStruct((N, DOUT), jnp.float32),
    )(deg_p, s2p, y2, b2.reshape(1, DOUT))

    return out
